# baseline (speedup 1.0000x reference)
import functools

import jax
import jax.numpy as jnp
import numpy as np
from jax.experimental import pallas as pl
from jax.experimental.pallas import tpu as pltpu

MAX_SCALE = 4
NUM_ANCHORS = 8
EPS = 1e-6
HBLK = 28
HALO = 4


def _shift_w(a, d):
    if d > 0:
        return jnp.concatenate([jnp.zeros_like(a[:, :d]), a[:, :-d]], axis=1)
    if d < 0:
        return jnp.concatenate([a[:, -d:], jnp.zeros_like(a[:, :-d])], axis=1)
    return a


def _fused_kernel(c_ref, anch_ref, wid_ref, x_ref, halo_ref, o_ref):
    xc = x_ref[0]
    top = halo_ref[0, 0, :HALO]
    bot = halo_ref[0, 0, HALO:]
    muc = jnp.maximum(xc, 0.0) + EPS
    mu = jnp.concatenate([top, muc, bot], axis=0)

    alpha = jnp.zeros_like(xc)
    k = mu + _shift_w(mu, 1) + _shift_w(mu, -1)
    for r in range(1, MAX_SCALE + 1):
        if r > 1:
            k = k + _shift_w(mu, r) + _shift_w(mu, -r)
        lo = HALO - r
        n = 2 * r + 1
        if r == 1:
            p2 = k[lo:lo + HBLK + 1] + k[lo + 1:lo + HBLK + 2]
            box = p2[:HBLK] + k[lo + 2:lo + 2 + HBLK]
        else:
            psz = k.shape[0] - lo - 1
            p2 = k[lo:lo + psz] + k[lo + 1:lo + 1 + psz]
            p4 = p2[:psz - 2] + p2[2:psz]
            if r == 2:
                box = p4[:HBLK] + k[lo + 4:lo + 4 + HBLK]
            elif r == 3:
                box = p4[:HBLK] + p2[4:4 + HBLK] + k[lo + 6:lo + 6 + HBLK]
            else:
                box = p4[:HBLK] + p4[4:4 + HBLK] + k[lo + 8:lo + 8 + HBLK]
        alpha = alpha + c_ref[r - 1] * jnp.log(box + EPS)

    acc = jnp.zeros_like(alpha)
    for a in range(NUM_ANCHORS):
        t = (alpha - anch_ref[a]) * wid_ref[a]
        m = jnp.minimum(1.0 - t, 1.0 + t)
        acc = acc + jnp.maximum(m, 0.0)
    o_ref[0] = xc + acc


@jax.jit
def kernel(x, scale_w, anchors, widths):
    b, h, w, c = x.shape
    nblk = h // HBLK

    sw = jax.nn.softmax(scale_w)
    log_r = jnp.log(jnp.asarray([2.0 * r + 1.0 for r in range(1, MAX_SCALE + 1)],
                                dtype=x.dtype))
    dev = log_r - jnp.sum(sw * log_r)
    var = jnp.sum(sw * dev * dev)
    coef = (sw * dev) / (var + EPS)

    zrow = jnp.zeros((b, HALO, w, c), dtype=x.dtype)
    halos = []
    for idx in range(nblk):
        top = zrow if idx == 0 else x[:, idx * HBLK - HALO:idx * HBLK]
        bot = zrow if idx == nblk - 1 else x[:, (idx + 1) * HBLK:(idx + 1) * HBLK + HALO]
        hx = jnp.concatenate([top, bot], axis=1)
        hmu = jnp.maximum(hx, 0.0) + EPS
        if idx == 0:
            hmu = hmu.at[:, :HALO].set(0.0)
        if idx == nblk - 1:
            hmu = hmu.at[:, HALO:].set(0.0)
        halos.append(hmu)
    halo = jnp.stack(halos, axis=1)

    anch = jnp.transpose(anchors).reshape(NUM_ANCHORS, 1, c)
    wid = jnp.transpose(widths).reshape(NUM_ANCHORS, 1, c)

    return pl.pallas_call(
        _fused_kernel,
        grid=(b, nblk),
        in_specs=[
            pl.BlockSpec(memory_space=pltpu.SMEM),
            pl.BlockSpec((NUM_ANCHORS, 1, c), lambda bi, hi: (0, 0, 0)),
            pl.BlockSpec((NUM_ANCHORS, 1, c), lambda bi, hi: (0, 0, 0)),
            pl.BlockSpec((1, HBLK, w, c), lambda bi, hi: (bi, hi, 0, 0)),
            pl.BlockSpec((1, 1, 2 * HALO, w, c), lambda bi, hi: (bi, hi, 0, 0, 0)),
        ],
        out_specs=pl.BlockSpec((1, HBLK, w, c), lambda bi, hi: (bi, hi, 0, 0)),
        out_shape=jax.ShapeDtypeStruct(x.shape, x.dtype),
    )(coef, anch, wid, x, halo)


# speedup vs baseline: 1.1289x; 1.1289x over previous
import functools

import jax
import jax.numpy as jnp
import numpy as np
from jax.experimental import pallas as pl
from jax.experimental.pallas import tpu as pltpu

MAX_SCALE = 4
NUM_ANCHORS = 8
EPS = 1e-6
HBLK = 56
HALO = 4


def _shift_w(a, d):
    if d > 0:
        return jnp.concatenate([jnp.zeros_like(a[:, :d]), a[:, :-d]], axis=1)
    if d < 0:
        return jnp.concatenate([a[:, -d:], jnp.zeros_like(a[:, :-d])], axis=1)
    return a


def _fused_kernel(c_ref, anch_ref, wid_ref, x_ref, halo_ref, o_ref):
    xc = x_ref[0]
    top = halo_ref[0, 0, :HALO]
    bot = halo_ref[0, 0, HALO:]
    muc = jnp.maximum(xc, 0.0) + EPS
    mu = jnp.concatenate([top, muc, bot], axis=0)

    alpha = jnp.zeros_like(xc)
    k = mu + _shift_w(mu, 1) + _shift_w(mu, -1)
    for r in range(1, MAX_SCALE + 1):
        if r > 1:
            k = k + _shift_w(mu, r) + _shift_w(mu, -r)
        lo = HALO - r
        n = 2 * r + 1
        if r == 1:
            p2 = k[lo:lo + HBLK + 1] + k[lo + 1:lo + HBLK + 2]
            box = p2[:HBLK] + k[lo + 2:lo + 2 + HBLK]
        else:
            psz = k.shape[0] - lo - 1
            p2 = k[lo:lo + psz] + k[lo + 1:lo + 1 + psz]
            p4 = p2[:psz - 2] + p2[2:psz]
            if r == 2:
                box = p4[:HBLK] + k[lo + 4:lo + 4 + HBLK]
            elif r == 3:
                box = p4[:HBLK] + p2[4:4 + HBLK] + k[lo + 6:lo + 6 + HBLK]
            else:
                box = p4[:HBLK] + p4[4:4 + HBLK] + k[lo + 8:lo + 8 + HBLK]
        alpha = alpha + c_ref[r - 1] * jnp.log2(box + EPS)

    acc = jnp.zeros_like(alpha)
    for a in range(NUM_ANCHORS):
        t = (alpha - anch_ref[a]) * wid_ref[a]
        acc = acc + jnp.minimum(jnp.abs(t), 1.0)
    o_ref[0] = (xc + float(NUM_ANCHORS)) - acc


@jax.jit
def kernel(x, scale_w, anchors, widths):
    b, h, w, c = x.shape
    nblk = h // HBLK

    sw = jax.nn.softmax(scale_w)
    log_r = jnp.log(jnp.asarray([2.0 * r + 1.0 for r in range(1, MAX_SCALE + 1)],
                                dtype=x.dtype))
    dev = log_r - jnp.sum(sw * log_r)
    var = jnp.sum(sw * dev * dev)
    coef = (sw * dev) * (float(np.log(2.0)) / (var + EPS))

    zrow = jnp.zeros((b, HALO, w, c), dtype=x.dtype)
    halos = []
    for idx in range(nblk):
        top = zrow if idx == 0 else x[:, idx * HBLK - HALO:idx * HBLK]
        bot = zrow if idx == nblk - 1 else x[:, (idx + 1) * HBLK:(idx + 1) * HBLK + HALO]
        hx = jnp.concatenate([top, bot], axis=1)
        hmu = jnp.maximum(hx, 0.0) + EPS
        if idx == 0:
            hmu = hmu.at[:, :HALO].set(0.0)
        if idx == nblk - 1:
            hmu = hmu.at[:, HALO:].set(0.0)
        halos.append(hmu)
    halo = jnp.stack(halos, axis=1)

    anch = jnp.transpose(anchors).reshape(NUM_ANCHORS, 1, c)
    wid = jnp.transpose(widths).reshape(NUM_ANCHORS, 1, c)

    return pl.pallas_call(
        _fused_kernel,
        grid=(b, nblk),
        in_specs=[
            pl.BlockSpec(memory_space=pltpu.SMEM),
            pl.BlockSpec((NUM_ANCHORS, 1, c), lambda bi, hi: (0, 0, 0)),
            pl.BlockSpec((NUM_ANCHORS, 1, c), lambda bi, hi: (0, 0, 0)),
            pl.BlockSpec((1, HBLK, w, c), lambda bi, hi: (bi, hi, 0, 0)),
            pl.BlockSpec((1, 1, 2 * HALO, w, c), lambda bi, hi: (bi, hi, 0, 0, 0)),
        ],
        out_specs=pl.BlockSpec((1, HBLK, w, c), lambda bi, hi: (bi, hi, 0, 0)),
        out_shape=jax.ShapeDtypeStruct(x.shape, x.dtype),
    )(coef, anch, wid, x, halo)
